# baseline (speedup 1.0000x reference)
import functools

import jax
import jax.numpy as jnp
from jax import lax
from jax.experimental import pallas as pl
from jax.experimental.pallas import tpu as pltpu
from jax.experimental.pallas import tpu_sc as plsc

_VOCAB = 1000000
_N_FIXED = 500000
_VEC = 32
_WORD_DIM = 64
_OUT_DIM = 64

_NC = 2
_NS = 16
_NW = _NC * _NS
_CHUNK = 128


def _sc_gather(B_L, n_chunks):
  per_w = B_L // _NW
  assert per_w == n_chunks * _CHUNK

  mesh = plsc.VectorSubcoreMesh(
      core_axis_name="c", subcore_axis_name="s",
      num_cores=_NC, num_subcores=_NS)

  out_t = jax.ShapeDtypeStruct((B_L, _VEC), jnp.float32)

  @functools.partial(
      pl.kernel,
      out_type=(out_t, out_t, out_t),
      mesh=mesh,
      compiler_params=pltpu.CompilerParams(use_tc_tiling_on_sc=False),
      scratch_types=[
          pltpu.VMEM((n_chunks, _CHUNK), jnp.int32),
          pltpu.VMEM((n_chunks, _CHUNK), jnp.int32),
          pltpu.VMEM((n_chunks, _CHUNK), jnp.int32),
          pltpu.VMEM((_CHUNK, _VEC), jnp.float32),
          pltpu.VMEM((_CHUNK, _VEC), jnp.float32),
          pltpu.VMEM((_CHUNK, _VEC), jnp.float32),
          pltpu.SemaphoreType.DMA,
          pltpu.SemaphoreType.DMA,
          pltpu.SemaphoreType.DMA,
      ],
  )
  def sc_fn(idxf_hbm, idxt_hbm, idxr_hbm, fixed_hbm, tuned_hbm, tv_hbm,
            rf_out, rt_out, tv_out,
            idxf_v, idxt_v, idxr_v, rf_b, rt_b, tv_b, s1, s2, s3):
    wid = lax.axis_index("s") * _NC + lax.axis_index("c")
    base = wid * per_w
    pltpu.sync_copy(idxf_hbm.at[wid], idxf_v)
    pltpu.sync_copy(idxt_hbm.at[wid], idxt_v)
    pltpu.sync_copy(idxr_hbm.at[wid], idxr_v)

    @pl.loop(0, n_chunks)
    def _chunk(j):
      c1 = pltpu.async_copy(fixed_hbm.at[idxf_v.at[j]], rf_b, s1)
      c2 = pltpu.async_copy(tuned_hbm.at[idxt_v.at[j]], rt_b, s2)
      c3 = pltpu.async_copy(tv_hbm.at[idxr_v.at[j]], tv_b, s3)
      c1.wait()
      c2.wait()
      c3.wait()
      row0 = base + j * _CHUNK
      pltpu.sync_copy(rf_b, rf_out.at[pl.ds(row0, _CHUNK)])
      pltpu.sync_copy(rt_b, rt_out.at[pl.ds(row0, _CHUNK)])
      pltpu.sync_copy(tv_b, tv_out.at[pl.ds(row0, _CHUNK)])

  return sc_fn


def _tc_project(B_L, tb):
  grid = B_L // tb

  def body(x_ref, rf_ref, rt_ref, tv_ref, w1_ref, w2_ref, out_ref):
    is_fixed = x_ref[...] < _N_FIXED
    first = jnp.where(is_fixed, rf_ref[...], rt_ref[...])
    out_ref[...] = (
        jnp.dot(first, w1_ref[...], preferred_element_type=jnp.float32)
        + jnp.dot(tv_ref[...], w2_ref[...], preferred_element_type=jnp.float32))

  return pl.pallas_call(
      body,
      grid=(grid,),
      in_specs=[
          pl.BlockSpec((tb, 1), lambda i: (i, 0)),
          pl.BlockSpec((tb, _VEC), lambda i: (i, 0)),
          pl.BlockSpec((tb, _VEC), lambda i: (i, 0)),
          pl.BlockSpec((tb, _VEC), lambda i: (i, 0)),
          pl.BlockSpec((_VEC, _OUT_DIM), lambda i: (0, 0)),
          pl.BlockSpec((_VEC, _OUT_DIM), lambda i: (0, 0)),
      ],
      out_specs=pl.BlockSpec((tb, _OUT_DIM), lambda i: (i, 0)),
      out_shape=jax.ShapeDtypeStruct((B_L, _OUT_DIM), jnp.float32),
  )


@jax.jit
def kernel(X, fixed_weight, tuned_weight, tuned_vector, W_lin):
  B, Lseq = X.shape
  B_L = B * Lseq
  per_w = B_L // _NW
  n_chunks = per_w // _CHUNK

  xf = X.reshape(-1)
  idx_f = jnp.minimum(xf, _N_FIXED - 1).reshape(_NW, n_chunks, _CHUNK)
  idx_t = jnp.maximum(xf - _N_FIXED, 0).reshape(_NW, n_chunks, _CHUNK)
  idx_r = xf.reshape(_NW, n_chunks, _CHUNK)

  rf, rt, tv = _sc_gather(B_L, n_chunks)(
      idx_f, idx_t, idx_r, fixed_weight, tuned_weight, tuned_vector)

  w1 = W_lin[:, :_VEC].T
  w2 = W_lin[:, _VEC:].T
  out = _tc_project(B_L, 2048)(xf.reshape(B_L, 1), rf, rt, tv, w1, w2)
  return out.reshape(B, Lseq, _OUT_DIM)
